# baseline (speedup 1.0000x reference)
import jax
import jax.numpy as jnp
from jax.experimental import pallas as pl
from jax.experimental.pallas import tpu as pltpu

_FANOUT = 16
_BLK = 8192


def _fused_body(e_ref, w0_ref, b0_ref, w1_ref, b1_ref, s_ref, out_ref):
    e = (e_ref[...] * 0.5).astype(jnp.bfloat16)
    b1h = b1_ref[...] * 0.5
    b0h = b0_ref[...] * 0.5
    th1 = jnp.tanh(
        jnp.dot(e, w1_ref[...].astype(jnp.bfloat16), preferred_element_type=jnp.float32) + b1h
    )
    p0 = 0.5 * jnp.tanh(
        jnp.dot(e, w0_ref[...].astype(jnp.bfloat16), preferred_element_type=jnp.float32) + b0h
    ) + 0.5
    p0h = jnp.dot(p0.astype(jnp.bfloat16), s_ref[...].astype(jnp.bfloat16), preferred_element_type=jnp.float32)
    out_ref[...] = p0h * th1 + p0h


def kernel(patient_embedding, y_true0, y_true1, W0, b0, W1, b1):
    B, D = patient_embedding.shape
    DIM0 = W0.shape[1]
    DIM1 = W1.shape[1]
    Sh = 0.5 * jnp.kron(
        jnp.eye(DIM0, dtype=jnp.float32), jnp.ones((1, _FANOUT), jnp.float32)
    )
    b0r = b0.reshape(1, DIM0)
    b1r = b1.reshape(1, DIM1)
    return pl.pallas_call(
        _fused_body,
        grid=(B // _BLK,),
        in_specs=[
            pl.BlockSpec((_BLK, D), lambda i: (i, 0)),
            pl.BlockSpec((D, DIM0), lambda i: (0, 0)),
            pl.BlockSpec((1, DIM0), lambda i: (0, 0)),
            pl.BlockSpec((D, DIM1), lambda i: (0, 0)),
            pl.BlockSpec((1, DIM1), lambda i: (0, 0)),
            pl.BlockSpec((DIM0, DIM1), lambda i: (0, 0)),
        ],
        out_specs=pl.BlockSpec((_BLK, DIM1), lambda i: (i, 0)),
        out_shape=jax.ShapeDtypeStruct((B, DIM1), jnp.float32),
        compiler_params=pltpu.CompilerParams(dimension_semantics=("parallel",)),
    )(patient_embedding, W0, b0r, W1, b1r, Sh)


# speedup vs baseline: 1.0684x; 1.0684x over previous
import jax
import jax.numpy as jnp
from jax.experimental import pallas as pl
from jax.experimental.pallas import tpu as pltpu

_FANOUT = 16
_BLK = 4096


def _fused_body(e_ref, w0_ref, b0_ref, w1_ref, b1_ref, s_ref, out_ref):
    e = (e_ref[...] * 0.5).astype(jnp.bfloat16)
    b1h = b1_ref[...] * 0.5
    b0h = b0_ref[...] * 0.5
    th1 = jnp.tanh(
        jnp.dot(e, w1_ref[...].astype(jnp.bfloat16), preferred_element_type=jnp.float32) + b1h
    )
    p0 = 0.5 * jnp.tanh(
        jnp.dot(e, w0_ref[...].astype(jnp.bfloat16), preferred_element_type=jnp.float32) + b0h
    ) + 0.5
    p0h = jnp.dot(p0.astype(jnp.bfloat16), s_ref[...].astype(jnp.bfloat16), preferred_element_type=jnp.float32)
    out_ref[...] = p0h * th1 + p0h


def kernel(patient_embedding, y_true0, y_true1, W0, b0, W1, b1):
    B, D = patient_embedding.shape
    DIM0 = W0.shape[1]
    DIM1 = W1.shape[1]
    Sh = 0.5 * jnp.kron(
        jnp.eye(DIM0, dtype=jnp.float32), jnp.ones((1, _FANOUT), jnp.float32)
    )
    b0r = b0.reshape(1, DIM0)
    b1r = b1.reshape(1, DIM1)
    return pl.pallas_call(
        _fused_body,
        grid=(B // _BLK,),
        in_specs=[
            pl.BlockSpec((_BLK, D), lambda i: (i, 0)),
            pl.BlockSpec((D, DIM0), lambda i: (0, 0)),
            pl.BlockSpec((1, DIM0), lambda i: (0, 0)),
            pl.BlockSpec((D, DIM1), lambda i: (0, 0)),
            pl.BlockSpec((1, DIM1), lambda i: (0, 0)),
            pl.BlockSpec((DIM0, DIM1), lambda i: (0, 0)),
        ],
        out_specs=pl.BlockSpec((_BLK, DIM1), lambda i: (i, 0)),
        out_shape=jax.ShapeDtypeStruct((B, DIM1), jnp.float32),
        compiler_params=pltpu.CompilerParams(dimension_semantics=("parallel",)),
    )(patient_embedding, W0, b0r, W1, b1r, Sh)
